# baseline (speedup 1.0000x reference)
import jax
import jax.numpy as jnp
from jax import lax
from jax.experimental import pallas as pl
from jax.experimental.pallas import tpu as pltpu
from jax.experimental.pallas import tpu_sc as plsc

_NUM_USERS = 500000
_LANES = 16


def _make_body(batch, emb, nw):
    bpw = batch // nw
    ngrp = bpw // _LANES

    def body(id_hbm, eu_hbm, ei_hbm, out_hbm, ids_v, tbuf, obuf, gsem, osem):
        nc = lax.axis_size("c")
        wid = lax.axis_index("s") * nc + lax.axis_index("c")
        base = wid * bpw

        pltpu.sync_copy(id_hbm.at[pl.ds(base, bpw)], ids_v)

        def fire(g):
            p = g % 3
            idv = ids_v[pl.ds(g * _LANES, _LANES)]
            eff = jnp.where(idv < _NUM_USERS, idv, idv - _NUM_USERS)
            for u in range(_LANES):
                s = idv[u]
                t = eff[u] >> 3

                @pl.when(s < _NUM_USERS)
                def _():
                    t8 = pl.multiple_of(t << 3, 8)
                    pltpu.async_copy(eu_hbm.at[pl.ds(t8, 8), :],
                                     tbuf.at[p, u], gsem)

                @pl.when(s >= _NUM_USERS)
                def _():
                    pltpu.async_copy(ei_hbm.at[t], tbuf.at[p, u], gsem)

        def drain_select(g):
            p = g % 3
            for u in range(_LANES):
                pltpu.make_async_copy(ei_hbm.at[0],
                                      tbuf.at[p, u], gsem).wait()
            idv = ids_v[pl.ds(g * _LANES, _LANES)]
            eff = jnp.where(idv < _NUM_USERS, idv, idv - _NUM_USERS)
            sub = eff & 7
            for u in range(_LANES):
                sb = sub[u]
                r = g * _LANES + u
                for cc in range(emb // _LANES):
                    obuf[r, pl.ds(cc * _LANES, _LANES)] = (
                        tbuf[p, u, sb, pl.ds(cc * _LANES, _LANES)])

        fire(0)
        fire(1)

        def pipe(g, carry):
            fire(g + 2)
            drain_select(g)
            return carry

        lax.fori_loop(0, ngrp - 2, pipe, 0)
        drain_select(ngrp - 2)
        drain_select(ngrp - 1)
        pltpu.sync_copy(obuf, out_hbm.at[pl.ds(base, bpw)])

    return body, bpw


def kernel(id, e_user, e_item):
    batch = id.shape[0]
    emb = e_user.shape[1]
    info = plsc.get_sparse_core_info()
    nw = info.num_cores * info.num_subcores
    ei3 = e_item.reshape(e_item.shape[0] // 8, 8, emb)
    body, bpw = _make_body(batch, emb, nw)
    mesh = plsc.VectorSubcoreMesh(core_axis_name="c", subcore_axis_name="s")
    f = pl.kernel(
        body,
        out_type=jax.ShapeDtypeStruct((batch, emb), jnp.float32),
        mesh=mesh,
        compiler_params=pltpu.CompilerParams(use_tc_tiling_on_sc=True),
        scratch_types=[
            pltpu.VMEM((bpw,), jnp.int32),
            pltpu.VMEM((3, _LANES, 8, emb), jnp.float32),
            pltpu.VMEM((bpw, emb), jnp.float32),
            pltpu.SemaphoreType.DMA,
            pltpu.SemaphoreType.DMA,
        ],
    )
    return f(id, e_user, ei3)


# speedup vs baseline: 1.0460x; 1.0460x over previous
import jax
import jax.numpy as jnp
from jax import lax
from jax.experimental import pallas as pl
from jax.experimental.pallas import tpu as pltpu
from jax.experimental.pallas import tpu_sc as plsc

_NUM_USERS = 500000
_LANES = 16


def _make_body(batch, emb, nw):
    bpw = batch // nw
    ngrp = bpw // _LANES

    def body(id_hbm, eu_hbm, ei_hbm, out_hbm, ids_v, tbuf, obuf, gsem, osem):
        nc = lax.axis_size("c")
        wid = lax.axis_index("s") * nc + lax.axis_index("c")
        base = wid * bpw

        pltpu.sync_copy(id_hbm.at[pl.ds(base, bpw)], ids_v)

        def fire(g):
            p = g % 3
            idv = ids_v[pl.ds(g * _LANES, _LANES)]
            eff = jnp.where(idv < _NUM_USERS, idv, idv - _NUM_USERS)
            for u in range(_LANES):
                s = idv[u]
                t = eff[u] >> 3

                @pl.when(s < _NUM_USERS)
                def _():
                    pltpu.async_copy(eu_hbm.at[t], tbuf.at[p, u], gsem)

                @pl.when(s >= _NUM_USERS)
                def _():
                    pltpu.async_copy(ei_hbm.at[t], tbuf.at[p, u], gsem)

        def drain_select(g):
            p = g % 3
            for u in range(_LANES):
                pltpu.make_async_copy(eu_hbm.at[0],
                                      tbuf.at[p, u], gsem).wait()
            idv = ids_v[pl.ds(g * _LANES, _LANES)]
            eff = jnp.where(idv < _NUM_USERS, idv, idv - _NUM_USERS)
            sub = eff & 7
            for u in range(_LANES):
                sb = sub[u]
                r = g * _LANES + u
                for cc in range(emb // _LANES):
                    obuf[r, pl.ds(cc * _LANES, _LANES)] = (
                        tbuf[p, u, sb, pl.ds(cc * _LANES, _LANES)])

        fire(0)
        fire(1)

        def pipe(g, carry):
            fire(g + 2)
            drain_select(g)
            return carry

        lax.fori_loop(0, ngrp - 2, pipe, 0)
        drain_select(ngrp - 2)
        drain_select(ngrp - 1)
        pltpu.sync_copy(obuf, out_hbm.at[pl.ds(base, bpw)])

    return body, bpw


def kernel(id, e_user, e_item):
    batch = id.shape[0]
    emb = e_user.shape[1]
    info = plsc.get_sparse_core_info()
    nw = info.num_cores * info.num_subcores
    eu3 = e_user.reshape(e_user.shape[0] // 8, 8, emb)
    ei3 = e_item.reshape(e_item.shape[0] // 8, 8, emb)
    body, bpw = _make_body(batch, emb, nw)
    mesh = plsc.VectorSubcoreMesh(core_axis_name="c", subcore_axis_name="s")
    f = pl.kernel(
        body,
        out_type=jax.ShapeDtypeStruct((batch, emb), jnp.float32),
        mesh=mesh,
        compiler_params=pltpu.CompilerParams(use_tc_tiling_on_sc=True),
        scratch_types=[
            pltpu.VMEM((bpw,), jnp.int32),
            pltpu.VMEM((3, _LANES, 8, emb), jnp.float32),
            pltpu.VMEM((bpw, emb), jnp.float32),
            pltpu.SemaphoreType.DMA,
            pltpu.SemaphoreType.DMA,
        ],
    )
    return f(id, eu3, ei3)
